# baseline (speedup 1.0000x reference)
import functools

import jax
import jax.numpy as jnp
from jax import lax
from jax.experimental import pallas as pl
from jax.experimental.pallas import tpu as pltpu
from jax.experimental.pallas import tpu_sc as plsc

N = 10000
E = 320000
D = 128
NC = 2
NS = 16
NW = NC * NS
K = 80
TCH = 4096
BCH = 32
BPAIR = BCH // 2
Q0 = 256
Q1 = 0
NBLK0 = Q0 // BCH
NBLK1 = Q1 // BCH
C1OFF = NS * Q0
EP = TCH * K
NP = 10240
RPT = NP // NS
BM = 2000


def _sc_body(with_counts, *refs):
    if with_counts:
        (a_hbm, src_hbm, dst_hbm, z2_hbm, z1_hbm, ones_hbm,
         g_hbm, cnt_hbm,
         src_v0, src_v1, dst_v0, dst_v1, rows0_v, rows1_v, ones_v,
         sem0, sem1, semi, acc_sh, cnt_sh) = refs
    else:
        (a_hbm, src_hbm, dst_hbm, z2_hbm,
         g_hbm,
         src_v0, src_v1, dst_v0, dst_v1, rows0_v, rows1_v,
         sem0, sem1, semi, acc_sh) = refs
    c = lax.axis_index("c")
    s = lax.axis_index("s")

    pltpu.sync_copy(z2_hbm, acc_sh.at[pl.ds(s * RPT, RPT)])
    if with_counts:
        pltpu.sync_copy(z1_hbm, cnt_sh.at[pl.ds(s * RPT, RPT)])
        pltpu.sync_copy(ones_hbm, ones_v)
    plsc.subcore_barrier()

    def run(nblk, base):
        if nblk == 0:
            return
        pltpu.sync_copy(src_hbm.at[pl.ds(base, BCH)], src_v0)
        pltpu.sync_copy(dst_hbm.at[pl.ds(base, BCH)], dst_v0)
        idx_bufs = ((src_v0, dst_v0), (src_v1, dst_v1))
        pltpu.async_copy(a_hbm.at[src_v0.at[0]], rows0_v, sem0)
        for b in range(nblk):
            sv, dv = idx_bufs[b % 2]
            nsv, ndv = idx_bufs[(b + 1) % 2]
            if b < nblk - 1:
                nxt = base + (b + 1) * BCH
                pltpu.async_copy(src_hbm.at[pl.ds(nxt, BCH)], nsv, semi)
                pltpu.async_copy(dst_hbm.at[pl.ds(nxt, BCH)], ndv, semi)

            def pair(t, carry, sv=sv, dv=dv):
                j0 = 2 * t
                j1 = j0 + 1
                pltpu.async_copy(a_hbm.at[sv.at[j1]], rows1_v, sem1)
                pltpu.make_async_copy(a_hbm.at[sv.at[j0]], rows0_v,
                                      sem0).wait()
                pltpu.sync_copy(rows0_v, acc_sh.at[dv.at[j0]], add=True)
                if with_counts:
                    pltpu.sync_copy(ones_v, cnt_sh.at[dv.at[j0]], add=True)

                @pl.when(t < BPAIR - 1)
                def _():
                    pltpu.async_copy(a_hbm.at[sv.at[j0 + 2]], rows0_v, sem0)

                pltpu.make_async_copy(a_hbm.at[sv.at[j1]], rows1_v,
                                      sem1).wait()
                pltpu.sync_copy(rows1_v, acc_sh.at[dv.at[j1]], add=True)
                if with_counts:
                    pltpu.sync_copy(ones_v, cnt_sh.at[dv.at[j1]], add=True)
                return carry

            lax.fori_loop(0, BPAIR, pair, 0)
            if b < nblk - 1:
                nxt = base + (b + 1) * BCH
                pltpu.make_async_copy(src_hbm.at[pl.ds(nxt, BCH)], nsv,
                                      semi).wait()
                pltpu.make_async_copy(dst_hbm.at[pl.ds(nxt, BCH)], ndv,
                                      semi).wait()
                pltpu.async_copy(a_hbm.at[nsv.at[0]], rows0_v, sem0)

    @pl.when(c == 0)
    def _():
        run(NBLK0, s * Q0)

    @pl.when(c == 1)
    def _():
        run(NBLK1, C1OFF + s * Q1)

    plsc.subcore_barrier()
    pltpu.sync_copy(acc_sh.at[pl.ds(s * RPT, RPT)],
                    g_hbm.at[c, pl.ds(s * RPT, RPT)])
    if with_counts:
        pltpu.sync_copy(cnt_sh.at[pl.ds(s * RPT, RPT)],
                        cnt_hbm.at[c, pl.ds(s * RPT, RPT)])


_MESH = plsc.VectorSubcoreMesh(core_axis_name="c", subcore_axis_name="s",
                               num_cores=NC, num_subcores=NS)

_sc_counts = pl.kernel(
    functools.partial(_sc_body, True),
    out_type=(jax.ShapeDtypeStruct((NC, NP, D), jnp.float32),
              jax.ShapeDtypeStruct((NC, NP), jnp.float32)),
    mesh=_MESH,
    scratch_types=[
        pltpu.VMEM((BCH, K), jnp.int32),
        pltpu.VMEM((BCH, K), jnp.int32),
        pltpu.VMEM((BCH, K), jnp.int32),
        pltpu.VMEM((BCH, K), jnp.int32),
        pltpu.VMEM((K, D), jnp.float32),
        pltpu.VMEM((K, D), jnp.float32),
        pltpu.VMEM((K,), jnp.float32),
        pltpu.SemaphoreType.DMA,
        pltpu.SemaphoreType.DMA,
        pltpu.SemaphoreType.DMA,
        pltpu.VMEM_SHARED((NP, D), jnp.float32),
        pltpu.VMEM_SHARED((NP,), jnp.float32),
    ],
)

_sc_plain = pl.kernel(
    functools.partial(_sc_body, False),
    out_type=jax.ShapeDtypeStruct((NC, NP, D), jnp.float32),
    mesh=_MESH,
    scratch_types=[
        pltpu.VMEM((BCH, K), jnp.int32),
        pltpu.VMEM((BCH, K), jnp.int32),
        pltpu.VMEM((BCH, K), jnp.int32),
        pltpu.VMEM((BCH, K), jnp.int32),
        pltpu.VMEM((K, D), jnp.float32),
        pltpu.VMEM((K, D), jnp.float32),
        pltpu.SemaphoreType.DMA,
        pltpu.SemaphoreType.DMA,
        pltpu.SemaphoreType.DMA,
        pltpu.VMEM_SHARED((NP, D), jnp.float32),
    ],
)


def _pre_body(x_ref, wlT_ref, wrT_ref, bl_ref, a_ref, b_ref):
    xb = x_ref[...]
    a_ref[...] = jnp.dot(xb, wlT_ref[...], preferred_element_type=jnp.float32)
    b_ref[...] = (jnp.dot(xb, wrT_ref[...], preferred_element_type=jnp.float32)
                  + bl_ref[...])


_pre = pl.pallas_call(
    _pre_body,
    grid=(N // BM,),
    in_specs=[
        pl.BlockSpec((BM, D), lambda i: (i, 0)),
        pl.BlockSpec((D, D), lambda i: (0, 0)),
        pl.BlockSpec((D, D), lambda i: (0, 0)),
        pl.BlockSpec((1, D), lambda i: (0, 0)),
    ],
    out_specs=[pl.BlockSpec((BM, D), lambda i: (i, 0)),
               pl.BlockSpec((BM, D), lambda i: (i, 0))],
    out_shape=[jax.ShapeDtypeStruct((N, D), jnp.float32)] * 2,
)


def _mid_body(g0_ref, g1_ref, c0_ref, c1_ref, b1_ref, wlT_ref, wrT_ref,
              bl_ref, a2_ref, b2_ref):
    cnt = jnp.maximum(c0_ref[...] + c1_ref[...], 1.0)
    h = jnp.maximum((g0_ref[...] + g1_ref[...]) / cnt + b1_ref[...], 0.0)
    a2_ref[...] = jnp.dot(h, wlT_ref[...], preferred_element_type=jnp.float32)
    b2_ref[...] = (jnp.dot(h, wrT_ref[...], preferred_element_type=jnp.float32)
                   + bl_ref[...])


_mid = pl.pallas_call(
    _mid_body,
    grid=(N // BM,),
    in_specs=[
        pl.BlockSpec((BM, D), lambda i: (i, 0)),
        pl.BlockSpec((BM, D), lambda i: (i, 0)),
        pl.BlockSpec((BM, 1), lambda i: (i, 0)),
        pl.BlockSpec((BM, 1), lambda i: (i, 0)),
        pl.BlockSpec((BM, D), lambda i: (i, 0)),
        pl.BlockSpec((D, D), lambda i: (0, 0)),
        pl.BlockSpec((D, D), lambda i: (0, 0)),
        pl.BlockSpec((1, D), lambda i: (0, 0)),
    ],
    out_specs=[pl.BlockSpec((BM, D), lambda i: (i, 0)),
               pl.BlockSpec((BM, D), lambda i: (i, 0))],
    out_shape=[jax.ShapeDtypeStruct((N, D), jnp.float32)] * 2,
)


def _post_body(g0_ref, g1_ref, c0_ref, c1_ref, b2_ref, o_ref):
    cnt = jnp.maximum(c0_ref[...] + c1_ref[...], 1.0)
    o_ref[...] = jnp.maximum(
        (g0_ref[...] + g1_ref[...]) / cnt + b2_ref[...], 0.0)


_post = pl.pallas_call(
    _post_body,
    grid=(N // BM,),
    in_specs=[
        pl.BlockSpec((BM, D), lambda i: (i, 0)),
        pl.BlockSpec((BM, D), lambda i: (i, 0)),
        pl.BlockSpec((BM, 1), lambda i: (i, 0)),
        pl.BlockSpec((BM, 1), lambda i: (i, 0)),
        pl.BlockSpec((BM, D), lambda i: (i, 0)),
    ],
    out_specs=pl.BlockSpec((BM, D), lambda i: (i, 0)),
    out_shape=jax.ShapeDtypeStruct((N, D), jnp.float32),
)


def kernel(x, edge_index, Wl1, bl1, Wr1, Wl2, bl2, Wr2):
    pad = EP - E
    src2 = jnp.concatenate(
        [edge_index[0], jnp.zeros((pad,), jnp.int32)]).reshape(TCH, K)
    dst2 = jnp.concatenate(
        [edge_index[1], jnp.full((pad,), N, jnp.int32)]).reshape(TCH, K)
    z2 = jnp.zeros((RPT, D), jnp.float32)
    z1 = jnp.zeros((RPT,), jnp.float32)
    ones = jnp.ones((K,), jnp.float32)

    a1, b1 = _pre(x, Wl1.T, Wr1.T, bl1.reshape(1, D))
    g1, cnt = _sc_counts(a1, src2, dst2, z2, z1, ones)
    c0 = cnt[0, :N].reshape(N, 1)
    c1 = cnt[1, :N].reshape(N, 1)
    a2, b2 = _mid(g1[0, :N], g1[1, :N], c0, c1, b1, Wl2.T, Wr2.T,
                  bl2.reshape(1, D))
    g2 = _sc_plain(a2, src2, dst2, z2)
    return _post(g2[0, :N], g2[1, :N], c0, c1, b2)


# speedup vs baseline: 1.3031x; 1.3031x over previous
import functools

import jax
import jax.numpy as jnp
from jax import lax
from jax.experimental import pallas as pl
from jax.experimental.pallas import tpu as pltpu
from jax.experimental.pallas import tpu_sc as plsc

N = 10000
E = 320000
D = 128
NC = 2
NS = 16
NW = NC * NS
K = 80
TCH = 4096
BCH = 32
BPAIR = BCH // 2
Q0 = 192
Q1 = 64
NBLK0 = Q0 // BCH
NBLK1 = Q1 // BCH
C1OFF = NS * Q0
EP = TCH * K
NP = 10240
RPT = NP // NS
BM = 2000


def _sc_body(with_counts, *refs):
    if with_counts:
        (a_hbm, src_hbm, dst_hbm, z2_hbm, z1_hbm, ones_hbm,
         g_hbm, cnt_hbm,
         src_v0, src_v1, dst_v0, dst_v1, rows0_v, rows1_v, ones_v,
         sem0, sem1, semi, acc_sh, cnt_sh) = refs
    else:
        (a_hbm, src_hbm, dst_hbm, z2_hbm,
         g_hbm,
         src_v0, src_v1, dst_v0, dst_v1, rows0_v, rows1_v,
         sem0, sem1, semi, acc_sh) = refs
    c = lax.axis_index("c")
    s = lax.axis_index("s")

    pltpu.sync_copy(z2_hbm, acc_sh.at[pl.ds(s * RPT, RPT)])
    if with_counts:
        pltpu.sync_copy(z1_hbm, cnt_sh.at[pl.ds(s * RPT, RPT)])
        pltpu.sync_copy(ones_hbm, ones_v)
    plsc.subcore_barrier()

    def run(nblk, base):
        if nblk == 0:
            return
        pltpu.sync_copy(src_hbm.at[pl.ds(base, BCH)], src_v0)
        pltpu.sync_copy(dst_hbm.at[pl.ds(base, BCH)], dst_v0)
        idx_bufs = ((src_v0, dst_v0), (src_v1, dst_v1))
        pltpu.async_copy(a_hbm.at[src_v0.at[0]], rows0_v, sem0)
        for b in range(nblk):
            sv, dv = idx_bufs[b % 2]
            nsv, ndv = idx_bufs[(b + 1) % 2]
            if b < nblk - 1:
                nxt = base + (b + 1) * BCH
                pltpu.async_copy(src_hbm.at[pl.ds(nxt, BCH)], nsv, semi)
                pltpu.async_copy(dst_hbm.at[pl.ds(nxt, BCH)], ndv, semi)

            def pair(t, carry, sv=sv, dv=dv):
                j0 = 2 * t
                j1 = j0 + 1
                pltpu.async_copy(a_hbm.at[sv.at[j1]], rows1_v, sem1)
                pltpu.make_async_copy(a_hbm.at[sv.at[j0]], rows0_v,
                                      sem0).wait()
                pltpu.sync_copy(rows0_v, acc_sh.at[dv.at[j0]], add=True)
                if with_counts:
                    pltpu.sync_copy(ones_v, cnt_sh.at[dv.at[j0]], add=True)

                @pl.when(t < BPAIR - 1)
                def _():
                    pltpu.async_copy(a_hbm.at[sv.at[j0 + 2]], rows0_v, sem0)

                pltpu.make_async_copy(a_hbm.at[sv.at[j1]], rows1_v,
                                      sem1).wait()
                pltpu.sync_copy(rows1_v, acc_sh.at[dv.at[j1]], add=True)
                if with_counts:
                    pltpu.sync_copy(ones_v, cnt_sh.at[dv.at[j1]], add=True)
                return carry

            lax.fori_loop(0, BPAIR, pair, 0)
            if b < nblk - 1:
                nxt = base + (b + 1) * BCH
                pltpu.make_async_copy(src_hbm.at[pl.ds(nxt, BCH)], nsv,
                                      semi).wait()
                pltpu.make_async_copy(dst_hbm.at[pl.ds(nxt, BCH)], ndv,
                                      semi).wait()
                pltpu.async_copy(a_hbm.at[nsv.at[0]], rows0_v, sem0)

    @pl.when(c == 0)
    def _():
        run(NBLK0, s * Q0)

    @pl.when(c == 1)
    def _():
        run(NBLK1, C1OFF + s * Q1)

    plsc.subcore_barrier()
    pltpu.sync_copy(acc_sh.at[pl.ds(s * RPT, RPT)],
                    g_hbm.at[c, pl.ds(s * RPT, RPT)])
    if with_counts:
        pltpu.sync_copy(cnt_sh.at[pl.ds(s * RPT, RPT)],
                        cnt_hbm.at[c, pl.ds(s * RPT, RPT)])


_MESH = plsc.VectorSubcoreMesh(core_axis_name="c", subcore_axis_name="s",
                               num_cores=NC, num_subcores=NS)

_sc_counts = pl.kernel(
    functools.partial(_sc_body, True),
    out_type=(jax.ShapeDtypeStruct((NC, NP, D), jnp.float32),
              jax.ShapeDtypeStruct((NC, NP), jnp.float32)),
    mesh=_MESH,
    scratch_types=[
        pltpu.VMEM((BCH, K), jnp.int32),
        pltpu.VMEM((BCH, K), jnp.int32),
        pltpu.VMEM((BCH, K), jnp.int32),
        pltpu.VMEM((BCH, K), jnp.int32),
        pltpu.VMEM((K, D), jnp.float32),
        pltpu.VMEM((K, D), jnp.float32),
        pltpu.VMEM((K,), jnp.float32),
        pltpu.SemaphoreType.DMA,
        pltpu.SemaphoreType.DMA,
        pltpu.SemaphoreType.DMA,
        pltpu.VMEM_SHARED((NP, D), jnp.float32),
        pltpu.VMEM_SHARED((NP,), jnp.float32),
    ],
)

_sc_plain = pl.kernel(
    functools.partial(_sc_body, False),
    out_type=jax.ShapeDtypeStruct((NC, NP, D), jnp.float32),
    mesh=_MESH,
    scratch_types=[
        pltpu.VMEM((BCH, K), jnp.int32),
        pltpu.VMEM((BCH, K), jnp.int32),
        pltpu.VMEM((BCH, K), jnp.int32),
        pltpu.VMEM((BCH, K), jnp.int32),
        pltpu.VMEM((K, D), jnp.float32),
        pltpu.VMEM((K, D), jnp.float32),
        pltpu.SemaphoreType.DMA,
        pltpu.SemaphoreType.DMA,
        pltpu.SemaphoreType.DMA,
        pltpu.VMEM_SHARED((NP, D), jnp.float32),
    ],
)


def _pre_body(x_ref, wlT_ref, wrT_ref, bl_ref, a_ref, b_ref):
    xb = x_ref[...]
    a_ref[...] = jnp.dot(xb, wlT_ref[...], preferred_element_type=jnp.float32)
    b_ref[...] = (jnp.dot(xb, wrT_ref[...], preferred_element_type=jnp.float32)
                  + bl_ref[...])


_pre = pl.pallas_call(
    _pre_body,
    grid=(N // BM,),
    in_specs=[
        pl.BlockSpec((BM, D), lambda i: (i, 0)),
        pl.BlockSpec((D, D), lambda i: (0, 0)),
        pl.BlockSpec((D, D), lambda i: (0, 0)),
        pl.BlockSpec((1, D), lambda i: (0, 0)),
    ],
    out_specs=[pl.BlockSpec((BM, D), lambda i: (i, 0)),
               pl.BlockSpec((BM, D), lambda i: (i, 0))],
    out_shape=[jax.ShapeDtypeStruct((N, D), jnp.float32)] * 2,
)


def _mid_body(g0_ref, g1_ref, c0_ref, c1_ref, b1_ref, wlT_ref, wrT_ref,
              bl_ref, a2_ref, b2_ref):
    cnt = jnp.maximum(c0_ref[...] + c1_ref[...], 1.0)
    h = jnp.maximum((g0_ref[...] + g1_ref[...]) / cnt + b1_ref[...], 0.0)
    a2_ref[...] = jnp.dot(h, wlT_ref[...], preferred_element_type=jnp.float32)
    b2_ref[...] = (jnp.dot(h, wrT_ref[...], preferred_element_type=jnp.float32)
                   + bl_ref[...])


_mid = pl.pallas_call(
    _mid_body,
    grid=(N // BM,),
    in_specs=[
        pl.BlockSpec((BM, D), lambda i: (i, 0)),
        pl.BlockSpec((BM, D), lambda i: (i, 0)),
        pl.BlockSpec((BM, 1), lambda i: (i, 0)),
        pl.BlockSpec((BM, 1), lambda i: (i, 0)),
        pl.BlockSpec((BM, D), lambda i: (i, 0)),
        pl.BlockSpec((D, D), lambda i: (0, 0)),
        pl.BlockSpec((D, D), lambda i: (0, 0)),
        pl.BlockSpec((1, D), lambda i: (0, 0)),
    ],
    out_specs=[pl.BlockSpec((BM, D), lambda i: (i, 0)),
               pl.BlockSpec((BM, D), lambda i: (i, 0))],
    out_shape=[jax.ShapeDtypeStruct((N, D), jnp.float32)] * 2,
)


def _post_body(g0_ref, g1_ref, c0_ref, c1_ref, b2_ref, o_ref):
    cnt = jnp.maximum(c0_ref[...] + c1_ref[...], 1.0)
    o_ref[...] = jnp.maximum(
        (g0_ref[...] + g1_ref[...]) / cnt + b2_ref[...], 0.0)


_post = pl.pallas_call(
    _post_body,
    grid=(N // BM,),
    in_specs=[
        pl.BlockSpec((BM, D), lambda i: (i, 0)),
        pl.BlockSpec((BM, D), lambda i: (i, 0)),
        pl.BlockSpec((BM, 1), lambda i: (i, 0)),
        pl.BlockSpec((BM, 1), lambda i: (i, 0)),
        pl.BlockSpec((BM, D), lambda i: (i, 0)),
    ],
    out_specs=pl.BlockSpec((BM, D), lambda i: (i, 0)),
    out_shape=jax.ShapeDtypeStruct((N, D), jnp.float32),
)


def kernel(x, edge_index, Wl1, bl1, Wr1, Wl2, bl2, Wr2):
    pad = EP - E
    src2 = jnp.concatenate(
        [edge_index[0], jnp.zeros((pad,), jnp.int32)]).reshape(TCH, K)
    dst2 = jnp.concatenate(
        [edge_index[1], jnp.full((pad,), N, jnp.int32)]).reshape(TCH, K)
    z2 = jnp.zeros((RPT, D), jnp.float32)
    z1 = jnp.zeros((RPT,), jnp.float32)
    ones = jnp.ones((K,), jnp.float32)

    a1, b1 = _pre(x, Wl1.T, Wr1.T, bl1.reshape(1, D))
    g1, cnt = _sc_counts(a1, src2, dst2, z2, z1, ones)
    c0 = cnt[0, :N].reshape(N, 1)
    c1 = cnt[1, :N].reshape(N, 1)
    a2, b2 = _mid(g1[0, :N], g1[1, :N], c0, c1, b1, Wl2.T, Wr2.T,
                  bl2.reshape(1, D))
    g2 = _sc_plain(a2, src2, dst2, z2)
    return _post(g2[0, :N], g2[1, :N], c0, c1, b2)
